# Initial kernel scaffold; baseline (speedup 1.0000x reference)
import functools

import jax
import jax.numpy as jnp
from jax import lax
from jax.experimental import pallas as pl
from jax.experimental.pallas import tpu as pltpu
from jax.experimental.pallas import tpu_sc as plsc

VOCAB = 1000000
EMB = 64
B = 16384
WIN = 10
NEG = 20
TN = NEG + 1

NC = 2
NS = 16
NW = NC * NS
EPW = B // NW
CB = 32
NCHUNK = EPW // CB
GG = 32


def _sc_scores_body(tn_idx_hbm, ctx_idx_hbm, t_emb, c_emb, out_hbm,
                    tn_idx_v, ctx_idx_v, tn_rows, ctx_rows, score_buf, sem):
    wid = lax.axis_index("s") * NC + lax.axis_index("c")

    def chunk_body(c, _):
        elem_base = wid * EPW + c * CB
        pltpu.sync_copy(tn_idx_hbm.at[pl.ds(elem_base * TN // GG, TN)],
                        tn_idx_v)
        pltpu.sync_copy(ctx_idx_hbm.at[pl.ds(elem_base * WIN // GG, WIN)],
                        ctx_idx_v)
        cps = []
        for g in range(TN):
            cps.append(pltpu.async_copy(
                t_emb.at[tn_idx_v.at[g]],
                tn_rows.at[pl.ds(g * GG, GG)], sem))
        for g in range(WIN):
            cps.append(pltpu.async_copy(
                c_emb.at[ctx_idx_v.at[g]],
                ctx_rows.at[pl.ds(g * GG, GG)], sem))
        for cp in cps:
            cp.wait()

        def elem_body(i, _):
            m = []
            for k in range(EMB // 16):
                acc = ctx_rows[i * WIN, pl.ds(k * 16, 16)]
                for w in range(1, WIN):
                    acc = acc + ctx_rows[i * WIN + w, pl.ds(k * 16, 16)]
                m.append(acc * (1.0 / WIN))
            for j in range(TN):
                d = m[0] * tn_rows[i * TN + j, pl.ds(0, 16)]
                for k in range(1, EMB // 16):
                    d = d + m[k] * tn_rows[i * TN + j, pl.ds(k * 16, 16)]
                s = jnp.sum(d)
                score_buf[i * TN + j] = -s if j == 0 else s
            return 0

        lax.fori_loop(0, CB, elem_body, 0)
        pltpu.sync_copy(score_buf, out_hbm.at[pl.ds(elem_base * TN, CB * TN)])
        return 0

    lax.fori_loop(0, NCHUNK, chunk_body, 0)


_sc_scores = functools.partial(
    pl.kernel,
    mesh=plsc.VectorSubcoreMesh(core_axis_name="c", subcore_axis_name="s"),
    out_type=jax.ShapeDtypeStruct((B * TN,), jnp.float32),
    scratch_types=[
        pltpu.VMEM((TN, GG), jnp.int32),
        pltpu.VMEM((WIN, GG), jnp.int32),
        pltpu.VMEM((CB * TN, EMB), jnp.float32),
        pltpu.VMEM((CB * WIN, EMB), jnp.float32),
        pltpu.VMEM((CB * TN,), jnp.float32),
        pltpu.SemaphoreType.DMA,
    ],
)(_sc_scores_body)


def _loss_body(s_ref, o_ref):
    x = s_ref[...]
    sp = jnp.maximum(x, 0.0) + jnp.log(1.0 + jnp.exp(-jnp.abs(x)))
    o_ref[0, 0] = jnp.sum(sp) * (1.0 / B)


_loss_call = pl.pallas_call(
    _loss_body,
    out_shape=jax.ShapeDtypeStruct((1, 1), jnp.float32),
)


@jax.jit
def kernel(contexts, targets, negatives, context_emb, target_emb):
    tn = jnp.concatenate([targets[:, None], negatives], axis=1)
    tn_idx = tn.reshape(B * TN // GG, GG).astype(jnp.int32)
    ctx_idx = contexts.reshape(B * WIN // GG, GG).astype(jnp.int32)
    scores = _sc_scores(tn_idx, ctx_idx, target_emb, context_emb)
    loss = _loss_call(scores.reshape(B * TN // 128, 128))
    return loss[0, 0]


# speedup vs baseline: 4.8243x; 4.8243x over previous
import functools

import jax
import jax.numpy as jnp
from jax import lax
from jax.experimental import pallas as pl
from jax.experimental.pallas import tpu as pltpu
from jax.experimental.pallas import tpu_sc as plsc

VOCAB = 1000000
EMB = 64
B = 16384
WIN = 10
NEG = 20
TN = NEG + 1

NC = 2
NS = 16
NW = NC * NS
EPW = B // NW
CB = 32
NCHUNK = EPW // CB
GG = 32


def _sc_scores_body(tn_idx_hbm, ctx_idx_hbm, t_emb, c_emb, out_hbm,
                    tn_idx_v, ctx_idx_v, tn_rows, ctx_rows, score_buf, sem):
    wid = lax.axis_index("s") * NC + lax.axis_index("c")

    def chunk_body(c, _):
        elem_base = wid * EPW + c * CB
        pltpu.sync_copy(tn_idx_hbm.at[pl.ds(elem_base * TN, CB * TN)],
                        tn_idx_v)
        pltpu.sync_copy(ctx_idx_hbm.at[pl.ds(elem_base * WIN, CB * WIN)],
                        ctx_idx_v)
        cps = []
        for g in range(CB * TN // GG):
            cps.append(pltpu.async_copy(
                t_emb.at[tn_idx_v.at[pl.ds(g * GG, GG)]],
                tn_rows.at[pl.ds(g * GG, GG)], sem))
        for g in range(CB * WIN // GG):
            cps.append(pltpu.async_copy(
                c_emb.at[ctx_idx_v.at[pl.ds(g * GG, GG)]],
                ctx_rows.at[pl.ds(g * GG, GG)], sem))
        for cp in cps:
            cp.wait()

        lane = lax.iota(jnp.int32, 16)
        last_mask = lane == 15

        def elem_body(i, _):
            m = []
            for k in range(EMB // 16):
                acc = ctx_rows[i * WIN, pl.ds(k * 16, 16)]
                for w in range(1, WIN):
                    acc = acc + ctx_rows[i * WIN + w, pl.ds(k * 16, 16)]
                m.append(acc * (1.0 / WIN))
            for j in range(TN):
                d = m[0] * tn_rows[i * TN + j, pl.ds(0, 16)]
                for k in range(1, EMB // 16):
                    d = d + m[k] * tn_rows[i * TN + j, pl.ds(k * 16, 16)]
                c = plsc.cumsum(d)
                val = -c if j == 0 else c
                pos = jnp.full((16,), i * TN + j, dtype=jnp.int32)
                plsc.store_scatter(score_buf, [pos], val, mask=last_mask)
            return 0

        lax.fori_loop(0, CB, elem_body, 0)
        pltpu.sync_copy(score_buf, out_hbm.at[pl.ds(elem_base * TN, CB * TN)])
        return 0

    lax.fori_loop(0, NCHUNK, chunk_body, 0)


_sc_scores = functools.partial(
    pl.kernel,
    mesh=plsc.VectorSubcoreMesh(core_axis_name="c", subcore_axis_name="s"),
    compiler_params=pltpu.CompilerParams(
        needs_layout_passes=False, use_tc_tiling_on_sc=False),
    out_type=jax.ShapeDtypeStruct((B * TN,), jnp.float32),
    scratch_types=[
        pltpu.VMEM((CB * TN,), jnp.int32),
        pltpu.VMEM((CB * WIN,), jnp.int32),
        pltpu.VMEM((CB * TN, EMB), jnp.float32),
        pltpu.VMEM((CB * WIN, EMB), jnp.float32),
        pltpu.VMEM((CB * TN,), jnp.float32),
        pltpu.SemaphoreType.DMA,
    ],
)(_sc_scores_body)


def _loss_body(s_ref, o_ref):
    x = s_ref[...]
    sp = jnp.maximum(x, 0.0) + jnp.log(1.0 + jnp.exp(-jnp.abs(x)))
    o_ref[0, 0] = jnp.sum(sp) * (1.0 / B)


_loss_call = pl.pallas_call(
    _loss_body,
    out_shape=jax.ShapeDtypeStruct((1, 1), jnp.float32),
    out_specs=pl.BlockSpec(memory_space=pltpu.SMEM),
)


@jax.jit
def kernel(contexts, targets, negatives, context_emb, target_emb):
    tn = jnp.concatenate([targets[:, None], negatives], axis=1)
    tn_idx = tn.reshape(B * TN).astype(jnp.int32)
    ctx_idx = contexts.reshape(B * WIN).astype(jnp.int32)
    scores = _sc_scores(tn_idx, ctx_idx, target_emb, context_emb)
    loss = _loss_call(scores.reshape(B * TN // 128, 128))
    return loss[0, 0]
